# Initial kernel scaffold; baseline (speedup 1.0000x reference)
import jax
import jax.numpy as jnp
from jax.experimental import pallas as pl


def kernel(x0, data, t, condition_mask):
    raise NotImplementedError("write your pallas kernel here")


# speedup vs baseline: 1.7399x; 1.7399x over previous
import functools

import jax
import jax.numpy as jnp
from jax.experimental import pallas as pl
from jax.experimental.pallas import tpu as pltpu


def _fm_kernel(x0_ref, dataq_ref, data_ref, t_ref, cm_ref, out_ref, *, S, D):
    x0 = x0_ref[0]
    data_q = dataq_ref[0]
    data = data_ref[0]
    t = t_ref[0, 0, 0]
    cm = cm_ref[...]

    QB = x0.shape[0]

    x0c = jnp.where(cm > 0.5, data_q, x0)

    dn = jnp.sum(data * data, axis=1, keepdims=True)
    yaug = jnp.concatenate([data, dn], axis=1)
    xaug = jnp.concatenate(
        [-2.0 * x0c, jnp.ones((QB, 1), jnp.float32)], axis=1)
    scores = jax.lax.dot_general(
        xaug, yaug, (((1,), (1,)), ((), ())),
        preferred_element_type=jnp.float32,
        precision=jax.lax.Precision.HIGHEST)

    m = jnp.min(scores, axis=-1, keepdims=True)
    col = jax.lax.broadcasted_iota(jnp.int32, (QB, S), 1)
    idx = jnp.min(jnp.where(scores <= m, col, S), axis=-1,
                  keepdims=True)

    onehot = (col == idx).astype(jnp.float32)
    nearest = jax.lax.dot_general(
        onehot, data, (((1,), (0,)), ((), ())),
        preferred_element_type=jnp.float32,
        precision=jax.lax.Precision.HIGHEST)

    xt = t * data_q + (1.0 - t) * x0c
    xt = jnp.where(cm > 0.5, data_q, xt)
    ut = data_q - x0c

    out_ref[0] = jnp.concatenate([xt, ut, nearest], axis=-1)


def kernel(x0, data, t, condition_mask):
    B, S, D = x0.shape
    QB = 512
    cmf = condition_mask.astype(jnp.float32).reshape(1, D)
    t3 = t.reshape(B, 1, 1)

    grid = (B, S // QB)
    out = pl.pallas_call(
        functools.partial(_fm_kernel, S=S, D=D),
        grid=grid,
        in_specs=[
            pl.BlockSpec((1, QB, D), lambda b, q: (b, q, 0)),
            pl.BlockSpec((1, QB, D), lambda b, q: (b, q, 0)),
            pl.BlockSpec((1, S, D), lambda b, q: (b, 0, 0)),
            pl.BlockSpec((1, 1, 1), lambda b, q: (b, 0, 0)),
            pl.BlockSpec((1, D), lambda b, q: (0, 0)),
        ],
        out_specs=pl.BlockSpec((1, QB, 3 * D), lambda b, q: (b, q, 0)),
        out_shape=jax.ShapeDtypeStruct((B, S, 3 * D), jnp.float32),
    )(x0, data, data, t3, cmf)
    return out


# speedup vs baseline: 3.7629x; 2.1628x over previous
import functools

import jax
import jax.numpy as jnp
from jax import lax
from jax.experimental import pallas as pl
from jax.experimental.pallas import tpu as pltpu
from jax.experimental.pallas import tpu_sc as plsc


def _fm_tc_kernel(x0_ref, dataq_ref, data_ref, t_ref, cm_ref,
                  out_ref, idx_ref, *, S, D):
    b = pl.program_id(0)
    x0 = x0_ref[0]
    data_q = dataq_ref[0]
    data = data_ref[0]
    t = t_ref[0, 0, 0]
    cm = cm_ref[...]

    QB = x0.shape[0]

    x0c = jnp.where(cm > 0.5, data_q, x0)

    dn = jnp.sum(data * data, axis=1, keepdims=True)
    yaug = jnp.concatenate([data, dn], axis=1)
    xaug = jnp.concatenate(
        [-2.0 * x0c, jnp.ones((QB, 1), jnp.float32)], axis=1)
    scores = jax.lax.dot_general(
        xaug, yaug, (((1,), (1,)), ((), ())),
        preferred_element_type=jnp.float32,
        precision=jax.lax.Precision.HIGHEST)

    m = jnp.min(scores, axis=-1, keepdims=True)
    col = jax.lax.broadcasted_iota(jnp.int32, (QB, S), 1)
    idx = jnp.min(jnp.where(scores <= m, col, S), axis=-1,
                  keepdims=True)
    idx_ref[0] = idx + b * S

    xt = t * data_q + (1.0 - t) * x0c
    xt = jnp.where(cm > 0.5, data_q, xt)
    ut = data_q - x0c
    out_ref[0] = jnp.concatenate([xt, ut], axis=-1)


def _gather_sc_kernel(table_hbm, idx_hbm, out_hbm, idx_v, rows_v, sem,
                      *, b_per_w, NC):
    wid = lax.axis_index("s") * NC + lax.axis_index("c")
    base = wid * b_per_w
    pltpu.sync_copy(idx_hbm.at[pl.ds(base, b_per_w)], idx_v)
    pltpu.async_copy(table_hbm.at[idx_v], rows_v, sem).wait()
    pltpu.sync_copy(rows_v, out_hbm.at[pl.ds(base, b_per_w)])


def kernel(x0, data, t, condition_mask):
    B, S, D = x0.shape
    QB = 512
    cmf = condition_mask.astype(jnp.float32).reshape(1, D)
    t3 = t.reshape(B, 1, 1)

    grid = (B, S // QB)
    xtut, idxg = pl.pallas_call(
        functools.partial(_fm_tc_kernel, S=S, D=D),
        grid=grid,
        in_specs=[
            pl.BlockSpec((1, QB, D), lambda b, q: (b, q, 0)),
            pl.BlockSpec((1, QB, D), lambda b, q: (b, q, 0)),
            pl.BlockSpec((1, S, D), lambda b, q: (b, 0, 0)),
            pl.BlockSpec((1, 1, 1), lambda b, q: (b, 0, 0)),
            pl.BlockSpec((1, D), lambda b, q: (0, 0)),
        ],
        out_specs=[
            pl.BlockSpec((1, QB, 2 * D), lambda b, q: (b, q, 0)),
            pl.BlockSpec((1, QB, 1), lambda b, q: (b, q, 0)),
        ],
        out_shape=[
            jax.ShapeDtypeStruct((B, S, 2 * D), jnp.float32),
            jax.ShapeDtypeStruct((B, S, 1), jnp.int32),
        ],
    )(x0, data, data, t3, cmf)

    N = B * S
    info = plsc.get_sparse_core_info()
    NC, NS = info.num_cores, info.num_subcores
    NW = NC * NS
    b_per_w = N // NW
    mesh = plsc.VectorSubcoreMesh(core_axis_name="c", subcore_axis_name="s")

    gather = functools.partial(
        pl.kernel,
        mesh=mesh,
        compiler_params=pltpu.CompilerParams(use_tc_tiling_on_sc=False),
        out_type=jax.ShapeDtypeStruct((N, D), jnp.float32),
        scratch_types=[
            pltpu.VMEM((b_per_w,), jnp.int32),
            pltpu.VMEM((b_per_w, D), jnp.float32),
            pltpu.SemaphoreType.DMA,
        ],
    )(functools.partial(_gather_sc_kernel, b_per_w=b_per_w, NC=NC))

    nearest = gather(data.reshape(N, D), idxg.reshape(N))
    return jnp.concatenate([xtut, nearest.reshape(B, S, D)], axis=-1)


# speedup vs baseline: 4.0957x; 1.0884x over previous
import functools

import jax
import jax.numpy as jnp
from jax import lax
from jax.experimental import pallas as pl
from jax.experimental.pallas import tpu as pltpu
from jax.experimental.pallas import tpu_sc as plsc


def _fm_tc_kernel(x0_ref, dataq_ref, data_ref, t_ref, cm_ref,
                  out_ref, idx_ref, *, S, D):
    b = pl.program_id(0)
    x0 = x0_ref[0]
    data_q = dataq_ref[0]
    data = data_ref[0]
    t = t_ref[0, 0, 0]
    cm = cm_ref[...]

    QB = x0.shape[0]

    x0c = jnp.where(cm > 0.5, data_q, x0)

    dn = jnp.sum(data * data, axis=1, keepdims=True)
    yaug = jnp.concatenate([data, dn], axis=1)
    xaug = jnp.concatenate(
        [-2.0 * x0c, jnp.ones((QB, 1), jnp.float32)], axis=1)
    scores = jax.lax.dot_general(
        xaug, yaug, (((1,), (1,)), ((), ())),
        preferred_element_type=jnp.float32,
        precision=jax.lax.Precision.HIGHEST)

    idx = jnp.argmin(scores, axis=-1).reshape(QB, 1)
    idx_ref[0] = idx + b * S

    xt = t * data_q + (1.0 - t) * x0c
    xt = jnp.where(cm > 0.5, data_q, xt)
    ut = data_q - x0c
    out_ref[0] = jnp.concatenate([xt, ut], axis=-1)


def _gather_sc_kernel(table_hbm, idx_hbm, out_hbm, idx_v, rows_v, sem,
                      *, b_per_w, NC):
    wid = lax.axis_index("s") * NC + lax.axis_index("c")
    base = wid * b_per_w
    pltpu.sync_copy(idx_hbm.at[pl.ds(base, b_per_w)], idx_v)
    pltpu.async_copy(table_hbm.at[idx_v], rows_v, sem).wait()
    pltpu.sync_copy(rows_v, out_hbm.at[pl.ds(base, b_per_w)])


def kernel(x0, data, t, condition_mask):
    B, S, D = x0.shape
    QB = 1024
    cmf = condition_mask.astype(jnp.float32).reshape(1, D)
    t3 = t.reshape(B, 1, 1)

    grid = (B, S // QB)
    xtut, idxg = pl.pallas_call(
        functools.partial(_fm_tc_kernel, S=S, D=D),
        grid=grid,
        in_specs=[
            pl.BlockSpec((1, QB, D), lambda b, q: (b, q, 0)),
            pl.BlockSpec((1, QB, D), lambda b, q: (b, q, 0)),
            pl.BlockSpec((1, S, D), lambda b, q: (b, 0, 0)),
            pl.BlockSpec((1, 1, 1), lambda b, q: (b, 0, 0)),
            pl.BlockSpec((1, D), lambda b, q: (0, 0)),
        ],
        out_specs=[
            pl.BlockSpec((1, QB, 2 * D), lambda b, q: (b, q, 0)),
            pl.BlockSpec((1, QB, 1), lambda b, q: (b, q, 0)),
        ],
        out_shape=[
            jax.ShapeDtypeStruct((B, S, 2 * D), jnp.float32),
            jax.ShapeDtypeStruct((B, S, 1), jnp.int32),
        ],
    )(x0, data, data, t3, cmf)

    N = B * S
    info = plsc.get_sparse_core_info()
    NC, NS = info.num_cores, info.num_subcores
    NW = NC * NS
    b_per_w = N // NW
    mesh = plsc.VectorSubcoreMesh(core_axis_name="c", subcore_axis_name="s")

    gather = functools.partial(
        pl.kernel,
        mesh=mesh,
        compiler_params=pltpu.CompilerParams(use_tc_tiling_on_sc=False),
        out_type=jax.ShapeDtypeStruct((N, D), jnp.float32),
        scratch_types=[
            pltpu.VMEM((b_per_w,), jnp.int32),
            pltpu.VMEM((b_per_w, D), jnp.float32),
            pltpu.SemaphoreType.DMA,
        ],
    )(functools.partial(_gather_sc_kernel, b_per_w=b_per_w, NC=NC))

    nearest = gather(data.reshape(N, D), idxg.reshape(N))
    return jnp.concatenate([xtut, nearest.reshape(B, S, D)], axis=-1)


# speedup vs baseline: 4.2156x; 1.0293x over previous
import functools

import jax
import jax.numpy as jnp
from jax import lax
from jax.experimental import pallas as pl
from jax.experimental.pallas import tpu as pltpu
from jax.experimental.pallas import tpu_sc as plsc


def _fm_tc_kernel(x0_ref, dataq_ref, data_ref, t_ref, cm_ref,
                  out_ref, idx_ref, *, S, D):
    b = pl.program_id(0)
    x0 = x0_ref[0]
    data_q = dataq_ref[0]
    data = data_ref[0]
    t = t_ref[0, 0, 0]
    cm = cm_ref[...]

    QB = x0.shape[0]

    x0c = jnp.where(cm > 0.5, data_q, x0)

    dn = jnp.sum(data * data, axis=1, keepdims=True)
    yaug = jnp.concatenate([data, dn], axis=1)
    xaug = jnp.concatenate(
        [-2.0 * x0c, jnp.ones((QB, 1), jnp.float32)], axis=1)
    scores = jax.lax.dot_general(
        xaug, yaug, (((1,), (1,)), ((), ())),
        preferred_element_type=jnp.float32,
        precision=jax.lax.Precision.HIGHEST)

    idx = jnp.argmin(scores, axis=-1).reshape(QB, 1)
    idx_ref[0] = (idx + b * S).T

    xt = t * data_q + (1.0 - t) * x0c
    xt = jnp.where(cm > 0.5, data_q, xt)
    ut = data_q - x0c
    out_ref[0] = jnp.concatenate([xt, ut], axis=-1)


def _gather_sc_kernel(table_hbm, idx_hbm, out_hbm, idx_v, rows_v, sem,
                      *, b_per_w, NC):
    wid = lax.axis_index("s") * NC + lax.axis_index("c")
    base = wid * b_per_w
    pltpu.sync_copy(idx_hbm.at[pl.ds(base, b_per_w)], idx_v)
    pltpu.async_copy(table_hbm.at[idx_v], rows_v, sem).wait()
    pltpu.sync_copy(rows_v, out_hbm.at[pl.ds(base, b_per_w)])


def kernel(x0, data, t, condition_mask):
    B, S, D = x0.shape
    QB = 1024
    cmf = condition_mask.astype(jnp.float32).reshape(1, D)
    t3 = t.reshape(B, 1, 1)

    grid = (B, S // QB)
    xtut, idxg = pl.pallas_call(
        functools.partial(_fm_tc_kernel, S=S, D=D),
        grid=grid,
        in_specs=[
            pl.BlockSpec((1, QB, D), lambda b, q: (b, q, 0)),
            pl.BlockSpec((1, QB, D), lambda b, q: (b, q, 0)),
            pl.BlockSpec((1, S, D), lambda b, q: (b, 0, 0)),
            pl.BlockSpec((1, 1, 1), lambda b, q: (b, 0, 0)),
            pl.BlockSpec((1, D), lambda b, q: (0, 0)),
        ],
        out_specs=[
            pl.BlockSpec((1, QB, 2 * D), lambda b, q: (b, q, 0)),
            pl.BlockSpec((1, 1, QB), lambda b, q: (b, 0, q)),
        ],
        out_shape=[
            jax.ShapeDtypeStruct((B, S, 2 * D), jnp.float32),
            jax.ShapeDtypeStruct((B, 1, S), jnp.int32),
        ],
    )(x0, data, data, t3, cmf)

    N = B * S
    info = plsc.get_sparse_core_info()
    NC, NS = info.num_cores, info.num_subcores
    NW = NC * NS
    b_per_w = N // NW
    mesh = plsc.VectorSubcoreMesh(core_axis_name="c", subcore_axis_name="s")

    gather = functools.partial(
        pl.kernel,
        mesh=mesh,
        compiler_params=pltpu.CompilerParams(use_tc_tiling_on_sc=False),
        out_type=jax.ShapeDtypeStruct((N, D), jnp.float32),
        scratch_types=[
            pltpu.VMEM((b_per_w,), jnp.int32),
            pltpu.VMEM((b_per_w, D), jnp.float32),
            pltpu.SemaphoreType.DMA,
        ],
    )(functools.partial(_gather_sc_kernel, b_per_w=b_per_w, NC=NC))

    nearest = gather(data.reshape(N, D), idxg.reshape(N))
    return jnp.concatenate([xtut, nearest.reshape(B, S, D)], axis=-1)
